# baseline (speedup 1.0000x reference)
import functools

import jax
import jax.numpy as jnp
from jax import lax
from jax.experimental import pallas as pl
from jax.experimental.pallas import tpu as pltpu
from jax.experimental.pallas import tpu_sc as plsc

SEQ = 50
BATCH = 1024
EMB = 16
TOTAL = SEQ * BATCH

NUM_CORES = 2
NUM_SUBCORES = 16
NW = NUM_CORES * NUM_SUBCORES
HALF = NW // 2
PER_W = TOTAL // HALF
CHUNK = 128
NCHUNK = PER_W // CHUNK

_mesh = plsc.VectorSubcoreMesh(core_axis_name="c", subcore_axis_name="s")


@functools.partial(
    pl.kernel,
    mesh=_mesh,
    out_type=(
        jax.ShapeDtypeStruct((TOTAL, EMB), jnp.float32),
        jax.ShapeDtypeStruct((TOTAL, EMB), jnp.float32),
    ),
    scratch_types=[
        pltpu.VMEM((NCHUNK, CHUNK), jnp.int32),
        pltpu.VMEM((PER_W, EMB), jnp.float32),
        pltpu.SemaphoreType.DMA,
    ],
    compiler_params=pltpu.CompilerParams(use_tc_tiling_on_sc=False),
)
def _dual_gather(prem_hbm, hypo_hbm, table_hbm, prem_out, hypo_out,
                 idx_v, rows_v, sem):
    wid = lax.axis_index("s") * NUM_CORES + lax.axis_index("c")
    is_prem = wid < HALF
    blk = jnp.where(is_prem, wid, wid - HALF)

    @pl.when(is_prem)
    def _():
        pltpu.sync_copy(prem_hbm.at[blk], idx_v)

    @pl.when(jnp.logical_not(is_prem))
    def _():
        pltpu.sync_copy(hypo_hbm.at[blk], idx_v)

    copies = [
        pltpu.async_copy(
            table_hbm.at[idx_v.at[j]],
            rows_v.at[pl.ds(j * CHUNK, CHUNK)],
            sem,
        )
        for j in range(NCHUNK)
    ]
    for c in copies:
        c.wait()

    @pl.when(is_prem)
    def _():
        pltpu.sync_copy(rows_v, prem_out.at[pl.ds(blk * PER_W, PER_W)])

    @pl.when(jnp.logical_not(is_prem))
    def _():
        pltpu.sync_copy(rows_v, hypo_out.at[pl.ds(blk * PER_W, PER_W)])


def kernel(premises_batch, hypotheses_batch, embedding_table):
    prem_idx = premises_batch.reshape(HALF, NCHUNK, CHUNK).astype(jnp.int32)
    hypo_idx = hypotheses_batch.reshape(HALF, NCHUNK, CHUNK).astype(jnp.int32)
    prem_rows, hypo_rows = _dual_gather(prem_idx, hypo_idx, embedding_table)
    return (
        prem_rows.reshape(SEQ, BATCH, EMB),
        hypo_rows.reshape(SEQ, BATCH, EMB),
    )
